# baseline (speedup 1.0000x reference)
import functools

import jax
import jax.numpy as jnp
from jax import lax
from jax.experimental import pallas as pl
from jax.experimental.pallas import tpu as pltpu
from jax.experimental.pallas import tpu_sc as plsc

BATCH = 4096
N_WORDS = 50
EMB_DIM = 64
TOTAL = BATCH * N_WORDS
NUM_CORES = 2
NUM_SUBCORES = 16
NW = NUM_CORES * NUM_SUBCORES
PER_W = TOTAL // NW
CHUNK = 128
NCHUNK = PER_W // CHUNK

_mesh = plsc.VectorSubcoreMesh(core_axis_name="c", subcore_axis_name="s")


@functools.partial(
    pl.kernel,
    mesh=_mesh,
    out_type=jax.ShapeDtypeStruct((TOTAL, EMB_DIM), jnp.float32),
    scratch_types=[
        pltpu.VMEM((NCHUNK, CHUNK), jnp.int32),
        pltpu.VMEM((CHUNK, EMB_DIM), jnp.float32),
        pltpu.SemaphoreType.DMA,
    ],
    compiler_params=pltpu.CompilerParams(use_tc_tiling_on_sc=False),
)
def _emb_lookup(idx_hbm, table_hbm, out_hbm, idx_v, rows_v, sem):
    c = lax.axis_index("c")
    s = lax.axis_index("s")
    wid = s * NUM_CORES + c
    pltpu.sync_copy(idx_hbm.at[wid], idx_v)
    base = wid * PER_W

    def step(j, carry):
        pltpu.async_copy(table_hbm.at[idx_v.at[j]], rows_v, sem).wait()
        pltpu.sync_copy(rows_v, out_hbm.at[pl.ds(base + j * CHUNK, CHUNK)])
        return carry

    lax.fori_loop(0, NCHUNK, step, 0)


def kernel(input_words, embeddings):
    idx = input_words.astype(jnp.int32).reshape(NW, NCHUNK, CHUNK)
    out = _emb_lookup(idx, embeddings)
    return out.reshape(BATCH, N_WORDS, EMB_DIM)


# speedup vs baseline: 1.0438x; 1.0438x over previous
import functools

import jax
import jax.numpy as jnp
from jax import lax
from jax.experimental import pallas as pl
from jax.experimental.pallas import tpu as pltpu
from jax.experimental.pallas import tpu_sc as plsc

BATCH = 4096
N_WORDS = 50
EMB_DIM = 64
TOTAL = BATCH * N_WORDS
NUM_CORES = 2
NUM_SUBCORES = 16
NW = NUM_CORES * NUM_SUBCORES
PER_W = TOTAL // NW
CHUNK = 128
NBUF = 5
ROUND = NBUF * CHUNK
ROUNDS = PER_W // ROUND
NCHUNK = PER_W // CHUNK

_mesh = plsc.VectorSubcoreMesh(core_axis_name="c", subcore_axis_name="s")


@functools.partial(
    pl.kernel,
    mesh=_mesh,
    out_type=jax.ShapeDtypeStruct((TOTAL, EMB_DIM), jnp.float32),
    scratch_types=[
        pltpu.VMEM((NCHUNK, CHUNK), jnp.int32),
        pltpu.VMEM((2, ROUND, EMB_DIM), jnp.float32),
        pltpu.SemaphoreType.DMA,
        pltpu.SemaphoreType.DMA,
    ],
    compiler_params=pltpu.CompilerParams(use_tc_tiling_on_sc=False),
)
def _emb_lookup(idx_hbm, table_hbm, out_hbm, idx_v, rows_v, gsem, osem):
    c = lax.axis_index("c")
    s = lax.axis_index("s")
    wid = s * NUM_CORES + c
    pltpu.sync_copy(idx_hbm.at[wid], idx_v)
    base = wid * PER_W

    def g_desc(g, bset, b):
        return pltpu.make_async_copy(
            table_hbm.at[idx_v.at[g * NBUF + b]],
            rows_v.at[bset, pl.ds(b * CHUNK, CHUNK)],
            gsem)

    def w_desc(g, bset):
        return pltpu.make_async_copy(
            rows_v.at[bset],
            out_hbm.at[pl.ds(base + g * ROUND, ROUND)],
            osem)

    def fire_gathers(g, bset):
        for b in range(NBUF):
            g_desc(g, bset, b).start()

    def wait_gathers(g, bset):
        for b in range(NBUF):
            g_desc(g, bset, b).wait()

    fire_gathers(0, 0)
    fire_gathers(1, 1)
    wait_gathers(0, 0)
    w_desc(0, 0).start()

    def body(g, carry):
        bset = lax.rem(g, 2)
        wait_gathers(g, bset)
        w_desc(g - 1, 1 - bset).wait()
        fire_gathers(g + 1, 1 - bset)
        w_desc(g, bset).start()
        return carry

    lax.fori_loop(1, ROUNDS - 1, body, 0)

    last = ROUNDS - 1
    bset = last % 2
    wait_gathers(last, bset)
    w_desc(last - 1, 1 - bset).wait()
    w_desc(last, bset).start()
    w_desc(last, bset).wait()


def kernel(input_words, embeddings):
    idx = input_words.astype(jnp.int32).reshape(NW, NCHUNK, CHUNK)
    out = _emb_lookup(idx, embeddings)
    return out.reshape(BATCH, N_WORDS, EMB_DIM)
